# baseline (speedup 1.0000x reference)
import functools

import jax
import jax.numpy as jnp
from jax import lax
from jax.experimental import pallas as pl
from jax.experimental.pallas import tpu as pltpu
from jax.experimental.pallas import tpu_sc as plsc

_N = 100000
_H = 128
_NW = 32
_RCHUNK = 128
_NCH = 25
_ROWS_PER_W = _RCHUNK * _NCH


@functools.partial(
    pl.kernel,
    mesh=plsc.VectorSubcoreMesh(core_axis_name="c", subcore_axis_name="s"),
    out_type=jax.ShapeDtypeStruct((_N, _H), jnp.float32),
    scratch_types=[
        pltpu.VMEM((_RCHUNK,), jnp.int32),
        pltpu.VMEM((_RCHUNK,), jnp.int32),
        pltpu.VMEM((_RCHUNK, _H), jnp.float32),
        pltpu.VMEM((_RCHUNK, _H), jnp.float32),
        pltpu.SemaphoreType.DMA,
        pltpu.SemaphoreType.DMA,
        pltpu.SemaphoreType.DMA,
        pltpu.SemaphoreType.DMA,
    ],
)
def _sc_embed(tab_hbm, out_hbm, idx0, idx1, buf0, buf1, sg0, sg1, so0, so1):
    wid = lax.axis_index("s") * 2 + lax.axis_index("c")
    row0 = wid * _ROWS_PER_W
    idxs = (idx0, idx1)
    bufs = (buf0, buf1)
    sg = (sg0, sg1)
    so = (so0, so1)
    lanes = lax.iota(jnp.int32, 16)

    def build_idx(i, b):
        base = row0 + i * _RCHUNK
        for g in range(_RCHUNK // 16):
            idxs[b][pl.ds(g * 16, 16)] = jnp.minimum(base + g * 16 + lanes, _N - 1)

    def gather(b):
        return pltpu.async_copy(tab_hbm.at[idxs[b]], bufs[b], sg[b])

    def put(b):
        return pltpu.async_copy(bufs[b], out_hbm.at[idxs[b]], so[b])

    h_g = [None, None]
    h_o = [None, None]
    build_idx(0, 0)
    h_g[0] = gather(0)
    for i in range(_NCH):
        b = i % 2
        if i + 1 < _NCH:
            b2 = (i + 1) % 2
            if h_o[b2] is not None:
                h_o[b2].wait()
            build_idx(i + 1, b2)
            h_g[b2] = gather(b2)
        h_g[b].wait()
        h_o[b] = put(b)
    h_o[(_NCH - 2) % 2].wait()
    h_o[(_NCH - 1) % 2].wait()


def kernel(table):
    return _sc_embed(table)


# speedup vs baseline: 2.9479x; 2.9479x over previous
import functools

import jax
import jax.numpy as jnp
from jax import lax
from jax.experimental import pallas as pl
from jax.experimental.pallas import tpu as pltpu
from jax.experimental.pallas import tpu_sc as plsc

_N = 100000
_H = 128
_WORDS = _N * _H
_NC = 2
_PER_C = _WORDS // _NC
_CHUNK = 400_000
_NCHUNK = _PER_C // _CHUNK
_NBUF = 4


@functools.partial(
    pl.kernel,
    mesh=plsc.ScalarSubcoreMesh(axis_name="c", num_cores=_NC),
    out_type=jax.ShapeDtypeStruct((_WORDS,), jnp.float32),
    scratch_types=(
        [pltpu.VMEM_SHARED((_CHUNK,), jnp.float32) for _ in range(_NBUF)]
        + [pltpu.SemaphoreType.DMA for _ in range(2 * _NBUF)]
    ),
)
def _sc_copy(tab_hbm, out_hbm, *scratch):
    bufs = scratch[:_NBUF]
    sin = scratch[_NBUF:2 * _NBUF]
    sout = scratch[2 * _NBUF:]
    base = lax.axis_index("c") * _PER_C

    def in_copy(i):
        return pltpu.async_copy(
            tab_hbm.at[pl.ds(base + i * _CHUNK, _CHUNK)],
            bufs[i % _NBUF],
            sin[i % _NBUF],
        )

    def out_copy(i):
        return pltpu.async_copy(
            bufs[i % _NBUF],
            out_hbm.at[pl.ds(base + i * _CHUNK, _CHUNK)],
            sout[i % _NBUF],
        )

    hin = [None] * _NBUF
    hout = {}
    out_waited = set()
    for j in range(min(_NBUF - 1, _NCHUNK)):
        hin[j % _NBUF] = in_copy(j)
    for i in range(_NCHUNK):
        b = i % _NBUF
        hin[b].wait()
        hout[i] = out_copy(i)
        j = i + _NBUF - 1
        if j < _NCHUNK:
            prev = j - _NBUF
            if prev >= 0:
                hout[prev].wait()
                out_waited.add(prev)
            hin[j % _NBUF] = in_copy(j)
    for i in range(_NCHUNK):
        if i not in out_waited:
            hout[i].wait()


def kernel(table):
    flat = table.reshape(_WORDS)
    return _sc_copy(flat).reshape(_N, _H)
